# baseline (speedup 1.0000x reference)
import jax
import jax.numpy as jnp
from jax import lax
from jax.experimental import pallas as pl
from jax.experimental.pallas import tpu as pltpu


def _make_body(actions: int):
    def _body(s_ref, a_ref, w1f_ref, w2_ref, b2_ref, o_ref):
        s = s_ref[...].astype(jnp.bfloat16)
        a = a_ref[...]
        iota = lax.broadcasted_iota(jnp.int32, (a.shape[0], actions), 1)
        onehot = (a == iota).astype(jnp.bfloat16)

        x = jnp.concatenate([s, onehot], axis=1)
        h = jnp.dot(x, w1f_ref[...], preferred_element_type=jnp.float32)
        h = jnp.maximum(h, 0.0).astype(jnp.bfloat16)

        out_t = lax.dot_general(w2_ref[...], h, (((0,), (1,)), ((), ())),
                                preferred_element_type=jnp.float32)
        o_ref[...] = jnp.swapaxes(out_t, 0, 1) + b2_ref[...]

    return _body


def kernel(s, a, w1, b1, w2, b2):
    T, S = s.shape
    H = w1.shape[1]
    O = w2.shape[1]
    A = w1.shape[0] - S

    b1 = jnp.reshape(b1, (1, H)).astype(jnp.float32)
    b2 = jnp.reshape(b2, (1, O)).astype(jnp.float32)
    w1f = jnp.concatenate([w1[:S], w1[S:] + b1], axis=0).astype(jnp.bfloat16)
    w2b = w2.astype(jnp.bfloat16)

    TM = 8192
    grid = (pl.cdiv(T, TM),)

    return pl.pallas_call(
        _make_body(A),
        out_shape=jax.ShapeDtypeStruct((T, O), jnp.float32),
        grid=grid,
        in_specs=[
            pl.BlockSpec((TM, S), lambda i: (i, 0)),
            pl.BlockSpec((TM, 1), lambda i: (i, 0)),
            pl.BlockSpec((S + A, H), lambda i: (0, 0)),
            pl.BlockSpec((H, O), lambda i: (0, 0)),
            pl.BlockSpec((1, O), lambda i: (0, 0)),
        ],
        out_specs=pl.BlockSpec((TM, O), lambda i: (i, 0)),
        compiler_params=pltpu.CompilerParams(
            dimension_semantics=("arbitrary",)),
    )(s, a, w1f, w2b, b2)


# speedup vs baseline: 1.0565x; 1.0565x over previous
import jax
import jax.numpy as jnp
from jax import lax
from jax.experimental import pallas as pl
from jax.experimental.pallas import tpu as pltpu


def _make_body(actions: int, chunk: int, n_chunks: int):
    def _body(s_ref, a_ref, w1f_ref, w2_ref, b2_ref, o_ref):
        for c in range(n_chunks):
            r0 = c * chunk
            s = s_ref[pl.ds(r0, chunk), :].astype(jnp.bfloat16)
            a = a_ref[pl.ds(r0, chunk), :]
            iota = lax.broadcasted_iota(jnp.int32, (chunk, actions), 1)
            onehot = (a == iota).astype(jnp.bfloat16)

            x = jnp.concatenate([s, onehot], axis=1)
            h = jnp.dot(x, w1f_ref[...], preferred_element_type=jnp.float32)
            h = jnp.maximum(h, 0.0).astype(jnp.bfloat16)

            out = jnp.dot(h, w2_ref[...], preferred_element_type=jnp.float32)
            o_ref[pl.ds(r0, chunk), :] = out + b2_ref[...]

    return _body


def kernel(s, a, w1, b1, w2, b2):
    T, S = s.shape
    H = w1.shape[1]
    O = w2.shape[1]
    A = w1.shape[0] - S

    b1 = jnp.reshape(b1, (1, H)).astype(jnp.float32)
    b2 = jnp.reshape(b2, (1, O)).astype(jnp.float32)
    w1f = jnp.concatenate([w1[:S], w1[S:] + b1], axis=0).astype(jnp.bfloat16)
    w2b = w2.astype(jnp.bfloat16)

    TM = 8192
    CHUNK = 4096
    grid = (pl.cdiv(T, TM),)

    return pl.pallas_call(
        _make_body(A, CHUNK, TM // CHUNK),
        out_shape=jax.ShapeDtypeStruct((T, O), jnp.float32),
        grid=grid,
        in_specs=[
            pl.BlockSpec((TM, S), lambda i: (i, 0)),
            pl.BlockSpec((TM, 1), lambda i: (i, 0)),
            pl.BlockSpec((S + A, H), lambda i: (0, 0)),
            pl.BlockSpec((H, O), lambda i: (0, 0)),
            pl.BlockSpec((1, O), lambda i: (0, 0)),
        ],
        out_specs=pl.BlockSpec((TM, O), lambda i: (i, 0)),
        compiler_params=pltpu.CompilerParams(
            dimension_semantics=("arbitrary",)),
    )(s, a, w1f, w2b, b2)


# speedup vs baseline: 1.0899x; 1.0316x over previous
import jax
import jax.numpy as jnp
from jax import lax
from jax.experimental import pallas as pl
from jax.experimental.pallas import tpu as pltpu


def _make_body(actions: int):
    def _body(s_ref, a_ref, w1f_ref, w2_ref, b2_ref, o_ref):
        s = s_ref[...].astype(jnp.bfloat16)
        a = a_ref[...]
        iota = lax.broadcasted_iota(jnp.int8, (a.shape[0], actions), 1)
        onehot = (a == iota).astype(jnp.bfloat16)

        x = jnp.concatenate([s, onehot], axis=1)
        h = jnp.dot(x, w1f_ref[...], preferred_element_type=jnp.float32)
        h = jnp.maximum(h.astype(jnp.bfloat16), jnp.bfloat16(0.0))

        out = jnp.dot(h, w2_ref[...], preferred_element_type=jnp.float32)
        o_ref[...] = out + b2_ref[...]

    return _body


def kernel(s, a, w1, b1, w2, b2):
    T, S = s.shape
    H = w1.shape[1]
    O = w2.shape[1]
    A = w1.shape[0] - S

    b1 = jnp.reshape(b1, (1, H)).astype(jnp.float32)
    b2 = jnp.reshape(b2, (1, O)).astype(jnp.float32)
    w1f = jnp.concatenate([w1[:S], w1[S:] + b1], axis=0).astype(jnp.bfloat16)
    w2b = w2.astype(jnp.bfloat16)

    TM = 8192
    grid = (pl.cdiv(T, TM),)

    return pl.pallas_call(
        _make_body(A),
        out_shape=jax.ShapeDtypeStruct((T, O), jnp.float32),
        grid=grid,
        in_specs=[
            pl.BlockSpec((TM, S), lambda i: (i, 0)),
            pl.BlockSpec((TM, 1), lambda i: (i, 0)),
            pl.BlockSpec((S + A, H), lambda i: (0, 0)),
            pl.BlockSpec((H, O), lambda i: (0, 0)),
            pl.BlockSpec((1, O), lambda i: (0, 0)),
        ],
        out_specs=pl.BlockSpec((TM, O), lambda i: (i, 0)),
        compiler_params=pltpu.CompilerParams(
            dimension_semantics=("arbitrary",)),
    )(s, a.astype(jnp.int8), w1f, w2b, b2)


# speedup vs baseline: 1.0938x; 1.0035x over previous
import jax
import jax.numpy as jnp
from jax import lax
from jax.experimental import pallas as pl
from jax.experimental.pallas import tpu as pltpu


def _make_body(actions: int):
    def _body(s_ref, a_ref, w1f_ref, w2_ref, b2_ref, o_ref):
        s = s_ref[...].astype(jnp.bfloat16)
        a = a_ref[...]
        iota = lax.broadcasted_iota(jnp.int8, (a.shape[0], actions), 1)
        onehot = (a == iota).astype(jnp.bfloat16)

        x = jnp.concatenate([s, onehot], axis=1)
        h = jnp.dot(x, w1f_ref[...], preferred_element_type=jnp.float32)
        h = jnp.maximum(h.astype(jnp.bfloat16), jnp.bfloat16(0.0))

        out = jnp.dot(h, w2_ref[...], preferred_element_type=jnp.float32)
        o_ref[...] = out + b2_ref[...]

    return _body


def kernel(s, a, w1, b1, w2, b2):
    T, S = s.shape
    H = w1.shape[1]
    O = w2.shape[1]
    A = w1.shape[0] - S

    b1 = jnp.reshape(b1, (1, H)).astype(jnp.float32)
    b2 = jnp.reshape(b2, (1, O)).astype(jnp.float32)
    w1f = jnp.concatenate([w1[:S], w1[S:] + b1], axis=0).astype(jnp.bfloat16)
    w2b = w2.astype(jnp.bfloat16)

    TM = 8192
    grid = (pl.cdiv(T, TM),)

    return pl.pallas_call(
        _make_body(A),
        out_shape=jax.ShapeDtypeStruct((T, O), jnp.float32),
        grid=grid,
        in_specs=[
            pl.BlockSpec((TM, S), lambda i: (i, 0)),
            pl.BlockSpec((TM, 1), lambda i: (i, 0)),
            pl.BlockSpec((S + A, H), lambda i: (0, 0)),
            pl.BlockSpec((H, O), lambda i: (0, 0)),
            pl.BlockSpec((1, O), lambda i: (0, 0)),
        ],
        out_specs=pl.BlockSpec((TM, O), lambda i: (i, 0)),
        compiler_params=pltpu.CompilerParams(
            dimension_semantics=("parallel",)),
    )(s, a.astype(jnp.int8), w1f, w2b, b2)


# speedup vs baseline: 1.1214x; 1.0253x over previous
import jax
import jax.numpy as jnp
from jax import lax
from jax.experimental import pallas as pl
from jax.experimental.pallas import tpu as pltpu


def _make_body(actions: int):
    def _body(s_ref, a_ref, w1f_ref, w2_ref, b2_ref, o_ref):
        s = s_ref[...].astype(jnp.bfloat16)
        a = a_ref[...]
        iota = lax.broadcasted_iota(jnp.int8, (a.shape[0], actions), 1)
        onehot = (a == iota).astype(jnp.bfloat16)

        h = jnp.dot(s, w1f_ref[pl.ds(0, 256), :],
                    preferred_element_type=jnp.float32)
        h = h + jnp.dot(onehot, w1f_ref[pl.ds(256, 16), :],
                        preferred_element_type=jnp.float32)
        h = jnp.maximum(h.astype(jnp.bfloat16), jnp.bfloat16(0.0))

        out = jnp.dot(h, w2_ref[...], preferred_element_type=jnp.float32)
        o_ref[...] = out + b2_ref[...]

    return _body


def kernel(s, a, w1, b1, w2, b2):
    T, S = s.shape
    H = w1.shape[1]
    O = w2.shape[1]
    A = w1.shape[0] - S

    b1 = jnp.reshape(b1, (1, H)).astype(jnp.float32)
    b2 = jnp.reshape(b2, (1, O)).astype(jnp.float32)
    w1f = jnp.concatenate([w1[:S], w1[S:] + b1], axis=0).astype(jnp.bfloat16)
    w2b = w2.astype(jnp.bfloat16)

    TM = 8192
    grid = (pl.cdiv(T, TM),)

    return pl.pallas_call(
        _make_body(A),
        out_shape=jax.ShapeDtypeStruct((T, O), jnp.float32),
        grid=grid,
        in_specs=[
            pl.BlockSpec((TM, S), lambda i: (i, 0)),
            pl.BlockSpec((TM, 1), lambda i: (i, 0)),
            pl.BlockSpec((S + A, H), lambda i: (0, 0)),
            pl.BlockSpec((H, O), lambda i: (0, 0)),
            pl.BlockSpec((1, O), lambda i: (0, 0)),
        ],
        out_specs=pl.BlockSpec((TM, O), lambda i: (i, 0)),
        compiler_params=pltpu.CompilerParams(
            dimension_semantics=("parallel",)),
    )(s, a.astype(jnp.int8), w1f, w2b, b2)


# speedup vs baseline: 1.1347x; 1.0119x over previous
import jax
import jax.numpy as jnp
from jax import lax
from jax.experimental import pallas as pl
from jax.experimental.pallas import tpu as pltpu


def _make_body(actions: int, s_dim: int):
    def _body(s_ref, a_ref, w1f_ref, w2_ref, b2_ref, o_ref):
        s = s_ref[...].astype(jnp.bfloat16)
        a = a_ref[...]
        iota = lax.broadcasted_iota(
            jnp.int32, (a.shape[0], actions), 1).astype(jnp.bfloat16)
        onehot = (a == iota).astype(jnp.bfloat16)

        h = jnp.dot(s, w1f_ref[pl.ds(0, s_dim), :],
                    preferred_element_type=jnp.float32)
        h = h + jnp.dot(onehot, w1f_ref[pl.ds(s_dim, actions), :],
                        preferred_element_type=jnp.float32)
        h = jnp.maximum(h.astype(jnp.bfloat16), jnp.bfloat16(0.0))

        out = jnp.dot(h, w2_ref[...], preferred_element_type=jnp.float32)
        o_ref[...] = out + b2_ref[...]

    return _body


def kernel(s, a, w1, b1, w2, b2):
    T, S = s.shape
    H = w1.shape[1]
    O = w2.shape[1]
    A = w1.shape[0] - S

    b1 = jnp.reshape(b1, (1, H)).astype(jnp.float32)
    b2 = jnp.reshape(b2, (1, O)).astype(jnp.float32)
    w1f = jnp.concatenate([w1[:S], w1[S:] + b1], axis=0).astype(jnp.bfloat16)
    w2b = w2.astype(jnp.bfloat16)

    TM = 8192
    grid = (pl.cdiv(T, TM),)

    return pl.pallas_call(
        _make_body(A, S),
        out_shape=jax.ShapeDtypeStruct((T, O), jnp.float32),
        grid=grid,
        in_specs=[
            pl.BlockSpec((TM, S), lambda i: (i, 0)),
            pl.BlockSpec((TM, 1), lambda i: (i, 0)),
            pl.BlockSpec((S + A, H), lambda i: (0, 0)),
            pl.BlockSpec((H, O), lambda i: (0, 0)),
            pl.BlockSpec((1, O), lambda i: (0, 0)),
        ],
        out_specs=pl.BlockSpec((TM, O), lambda i: (i, 0)),
        compiler_params=pltpu.CompilerParams(
            dimension_semantics=("parallel",)),
    )(s, a.astype(jnp.bfloat16), w1f, w2b, b2)
